# Initial kernel scaffold; baseline (speedup 1.0000x reference)
import jax
import jax.numpy as jnp
from jax.experimental import pallas as pl


def kernel(x, edge_index, edge_attr, batch, W1, att_src1, att_dst1, We1, att_edge1, b1, W2, att_src2, att_dst2, We2, att_edge2, b2):
    raise NotImplementedError("write your pallas kernel here")


# speedup vs baseline: 1.3673x; 1.3673x over previous
import jax
import jax.numpy as jnp
from jax.experimental import pallas as pl


def _leaky(x):
    return jax.nn.leaky_relu(x, 0.2)


def kernel(x, edge_index, edge_attr, batch,
           W1, att_src1, att_dst1, We1, att_edge1, b1,
           W2, att_src2, att_dst2, We2, att_edge2, b2):
    N, D = x.shape
    E = edge_index.shape[1]
    DE = edge_attr.shape[1]
    H1, C1 = att_src1.shape[1], att_src1.shape[2]
    C2 = att_src2.shape[2]
    NG = 64
    src, dst = edge_index[0], edge_index[1]
    seg = jax.ops.segment_sum

    W1r = W1.reshape(D, H1, C1)
    Ms1 = jnp.einsum('dhc,hc->dh', W1r, att_src1[0])
    Md1 = jnp.einsum('dhc,hc->dh', W1r, att_dst1[0])
    Me1 = jnp.einsum('dhc,hc->dh', We1.reshape(DE, H1, C1), att_edge1[0])
    a_src = x @ Ms1
    a_dst = x @ Md1
    a_edge = edge_attr @ Me1
    ones = jnp.ones((E,), jnp.float32)
    deg = seg(ones, dst, num_segments=N)
    s_ae = seg(a_edge, dst, num_segments=N)
    a_edge_loop = s_ae / jnp.clip(deg, 1.0)[:, None]
    aexp = jnp.exp(_leaky(a_src[src] + a_dst[dst] + a_edge))
    asum = seg(aexp, dst, num_segments=N)
    aexp_loop = jnp.exp(_leaky(a_src + a_dst + a_edge_loop))
    asumT = asum + aexp_loop
    w = aexp / (asumT[dst] + 1e-16)
    wl = aexp_loop / (asumT + 1e-16)
    z = seg(w[:, :, None] * x[src][:, None, :], dst, num_segments=N)
    z = z + wl[:, :, None] * x[:, None, :]
    h1 = jax.nn.elu(jnp.einsum('nhd,dhc->nhc', z, W1r).reshape(N, H1 * C1) + b1)

    a_src2 = h1 @ (W2 @ att_src2[0, 0])
    a_dst2 = h1 @ (W2 @ att_dst2[0, 0])
    a_edge2 = edge_attr @ (We2 @ att_edge2[0, 0])
    s_ae2 = seg(a_edge2, dst, num_segments=N)
    ael2 = s_ae2 / jnp.clip(deg, 1.0)
    hp = h1 @ W2
    aexp2 = jnp.exp(_leaky(a_src2[src] + a_dst2[dst] + a_edge2))
    asum2 = seg(aexp2, dst, num_segments=N)
    ael_exp = jnp.exp(_leaky(a_src2 + a_dst2 + ael2))
    asum2T = asum2 + ael_exp
    w2 = aexp2 / (asum2T[dst] + 1e-16)
    wl2 = ael_exp / (asum2T + 1e-16)
    z2 = seg(w2[:, None] * hp[src], dst, num_segments=N) + wl2[:, None] * hp
    h2 = jax.nn.elu(z2 + b2)
    cnt = seg(jnp.ones((N,), h2.dtype), batch, num_segments=NG)
    pooled = seg(h2, batch, num_segments=NG) / jnp.clip(cnt, 1.0)[:, None]
    return pooled


# speedup vs baseline: 3.3966x; 2.4841x over previous
import functools

import jax
import jax.numpy as jnp
from jax import lax
from jax.experimental import pallas as pl
from jax.experimental.pallas import tpu as pltpu
from jax.experimental.pallas import tpu_sc as plsc

N = 10000
E = 320000
D = 128
DE = 16
H1 = 8
C1 = 256
C2 = 256
NG = 64

NTILES = 32
EPT = E // NTILES
CH = 2000

_MESH = plsc.VectorSubcoreMesh(core_axis_name="c", subcore_axis_name="s")


def _take16(v, idx):
    dn = lax.GatherDimensionNumbers(
        offset_dims=(), collapsed_slice_dims=(0,), start_index_map=(0,))
    return lax.gather(v, idx[:, None], dimension_numbers=dn, slice_sizes=(1,),
                      mode=lax.GatherScatterMode.PROMISE_IN_BOUNDS)


def _seg_scatter_add(acc_refs, keys, vals):
    lane = lax.iota(jnp.int32, 16)
    sk, perm = plsc.sort_key_val(keys, lane)
    shift = jnp.minimum(lane + 1, 15)
    nxt = _take16(sk, shift)
    is_last = (lane == 15) | (sk != nxt)
    prv = jnp.maximum(lane - 1, 0)
    is_first = (lane == 0) | (sk != _take16(sk, prv))
    sidx = plsc.cummax(jnp.where(is_first, lane, -1))
    pidx = jnp.maximum(sidx - 1, 0)
    first_seg = sidx == 0
    for acc, val in zip(acc_refs, vals):
        cs = jnp.cumsum(_take16(val, perm))
        pbc = jnp.where(first_seg, jnp.float32(0.0), _take16(cs, pidx))
        plsc.addupdate_scatter(acc, [sk], cs - pbc, mask=is_last)


def _sc_edge_stats(src, dst, ae, tab_src, tab_dst):
    @functools.partial(
        pl.kernel,
        mesh=_MESH,
        compiler_params=pltpu.CompilerParams(use_tc_tiling_on_sc=False,
                                             needs_layout_passes=False),
        out_type=(
            jax.ShapeDtypeStruct((E,), jnp.float32),
            jax.ShapeDtypeStruct((NTILES, 3, N), jnp.float32),
        ),
        scratch_types=[
            pltpu.VMEM((N,), jnp.float32),
            pltpu.VMEM((N,), jnp.float32),
            pltpu.VMEM((N,), jnp.float32),
            pltpu.VMEM((N,), jnp.float32),
            pltpu.VMEM((N,), jnp.float32),
            pltpu.VMEM((CH,), jnp.int32),
            pltpu.VMEM((CH,), jnp.int32),
            pltpu.VMEM((CH,), jnp.float32),
            pltpu.VMEM((CH,), jnp.float32),
        ],
    )
    def k(src_hbm, dst_hbm, ae_hbm, ts_hbm, td_hbm, ax_hbm, part_hbm,
          tab_s, tab_d, acc_a, acc_e, acc_d, src_b, dst_b, ae_b, ax_b):
        wid = lax.axis_index("s") * 2 + lax.axis_index("c")
        ebase = wid * EPT
        zf = jnp.zeros((16,), jnp.float32)

        @pl.loop(0, N, step=16)
        def _(i):
            acc_a[pl.ds(i, 16)] = zf
            acc_e[pl.ds(i, 16)] = zf
            acc_d[pl.ds(i, 16)] = zf

        pltpu.sync_copy(ts_hbm, tab_s)
        pltpu.sync_copy(td_hbm, tab_d)
        onef = jnp.ones((16,), jnp.float32)

        @pl.loop(0, EPT, step=CH)
        def _(cb):
            pltpu.sync_copy(src_hbm.at[pl.ds(ebase + cb, CH)], src_b)
            pltpu.sync_copy(dst_hbm.at[pl.ds(ebase + cb, CH)], dst_b)
            pltpu.sync_copy(ae_hbm.at[pl.ds(ebase + cb, CH)], ae_b)

            @pl.loop(0, CH, step=16)
            def _(v):
                sidx = src_b[pl.ds(v, 16)]
                didx = dst_b[pl.ds(v, 16)]
                gs = plsc.load_gather(tab_s, [sidx])
                gd = plsc.load_gather(tab_d, [didx])
                ae_v = ae_b[pl.ds(v, 16)]
                al = gs + gd + ae_v
                al = jnp.where(al >= 0.0, al, al * jnp.float32(0.2))
                ax = jnp.exp(al)
                ax_b[pl.ds(v, 16)] = ax
                _seg_scatter_add((acc_a, acc_e, acc_d), didx,
                                 (ax, ae_v, onef))

            pltpu.sync_copy(ax_b, ax_hbm.at[pl.ds(ebase + cb, CH)])

        pltpu.sync_copy(acc_a, part_hbm.at[wid, 0])
        pltpu.sync_copy(acc_e, part_hbm.at[wid, 1])
        pltpu.sync_copy(acc_d, part_hbm.at[wid, 2])

    return k(src, dst, ae, tab_src, tab_dst)


WSZ2 = 320
NPAD2 = WSZ2 * NTILES
MG = 64
CHB = 2000
_SC_PARAMS = pltpu.CompilerParams(use_tc_tiling_on_sc=False,
                                  needs_layout_passes=False)


def _sc_agg2(src, dst, aexp2, recip2pad, hp):
    @functools.partial(
        pl.kernel,
        mesh=_MESH,
        compiler_params=_SC_PARAMS,
        out_type=jax.ShapeDtypeStruct((NPAD2, C2), jnp.float32),
        scratch_types=[
            pltpu.VMEM((WSZ2, C2), jnp.float32),
            pltpu.VMEM((WSZ2,), jnp.float32),
            pltpu.VMEM((CHB,), jnp.int32),
            pltpu.VMEM((CHB,), jnp.int32),
            pltpu.VMEM((CHB,), jnp.float32),
            pltpu.VMEM((CHB + 2 * MG,), jnp.int32),
            pltpu.VMEM((CHB + 2 * MG,), jnp.int32),
            pltpu.VMEM((CHB + 2 * MG,), jnp.float32),
            pltpu.VMEM((CHB + 2 * MG,), jnp.float32),
            pltpu.VMEM((MG, C2), jnp.float32),
            pltpu.SemaphoreType.DMA,
        ],
    )
    def k(src_hbm, dst_hbm, ax_hbm, rc_hbm, hp_hbm, z_hbm,
          zf, rc_w, src_b, dst_b, ax_b, m_src, m_dloc, m_ax, m_w, rows, sem):
        wid = lax.axis_index("s") * 2 + lax.axis_index("c")
        lo = wid * WSZ2
        zf16 = jnp.zeros((16,), jnp.float32)
        zi16 = jnp.zeros((16,), jnp.int32)

        @pl.loop(0, WSZ2)
        def _(r):
            @pl.loop(0, C2, step=16)
            def _(q):
                zf.at[r][pl.ds(q, 16)] = zf16

        pltpu.sync_copy(rc_hbm.at[pl.ds(lo, WSZ2)], rc_w)

        @pl.loop(0, E, step=CHB)
        def _(cb):
            pltpu.sync_copy(src_hbm.at[pl.ds(cb, CHB)], src_b)
            pltpu.sync_copy(dst_hbm.at[pl.ds(cb, CHB)], dst_b)
            pltpu.sync_copy(ax_hbm.at[pl.ds(cb, CHB)], ax_b)

            def scan_body(v, ptr):
                dvec = dst_b[pl.ds(v * 16, 16)]
                svec = src_b[pl.ds(v * 16, 16)]
                avec = ax_b[pl.ds(v * 16, 16)]
                m = (dvec >= lo) & (dvec < lo + WSZ2)
                plsc.store_compressed(m_src.at[pl.ds(ptr, 16)], svec, mask=m)
                plsc.store_compressed(m_dloc.at[pl.ds(ptr, 16)], dvec - lo, mask=m)
                plsc.store_compressed(m_ax.at[pl.ds(ptr, 16)], avec, mask=m)
                return ptr + jnp.sum(m.astype(jnp.int32))

            ptr = lax.fori_loop(0, CHB // 16, scan_body, jnp.int32(0))

            t0 = (ptr // 16) * 16
            rel = ptr - t0
            lane0 = lax.iota(jnp.int32, 16)
            keep = lane0 < rel
            m_src[pl.ds(t0, 16)] = jnp.where(keep, m_src[pl.ds(t0, 16)], 0)
            m_dloc[pl.ds(t0, 16)] = jnp.where(keep, m_dloc[pl.ds(t0, 16)], 0)
            m_ax[pl.ds(t0, 16)] = jnp.where(
                keep, m_ax[pl.ds(t0, 16)], jnp.float32(0.0))

            @pl.loop(16, MG + 16, step=16)
            def _(t):
                m_src[pl.ds(t0 + t, 16)] = zi16
                m_dloc[pl.ds(t0 + t, 16)] = zi16

            def wbody(v, _):
                dl = m_dloc[pl.ds(v * 16, 16)]
                av = m_ax[pl.ds(v * 16, 16)]
                m_w[pl.ds(v * 16, 16)] = av * plsc.load_gather(rc_w, [dl])
                return 0

            nv = (ptr + 15) // 16
            lax.fori_loop(0, nv, wbody, 0)

            def gbody(i, _):
                g = i * MG
                pltpu.async_copy(hp_hbm.at[m_src.at[pl.ds(g, MG)]], rows,
                                 sem).wait()

                def sgbody(v, _):
                    base = g + v * 16
                    dl_v = m_dloc[pl.ds(base, 16)]
                    w_v = m_w[pl.ds(base, 16)]
                    for j in range(16):
                        @pl.when(base + j < ptr)
                        def _():
                            zrow = zf.at[dl_v[j]]
                            xrow = rows.at[v * 16 + j]
                            wsc = w_v[j]
                            for q in range(C2 // 16):
                                plsc.addupdate(zrow.at[pl.ds(q * 16, 16)],
                                               xrow[pl.ds(q * 16, 16)] * wsc)
                    return 0

                nv = jnp.minimum((ptr - g + 15) // 16, MG // 16)
                lax.fori_loop(0, nv, sgbody, 0)
                return 0

            ng = (ptr + MG - 1) // MG
            lax.fori_loop(0, ng, gbody, 0)

        pltpu.sync_copy(zf, z_hbm.at[pl.ds(lo, WSZ2)])

    return k(src, dst, aexp2, recip2pad, hp)


WSZ1 = 105
NPASS1 = 3
NPAD1 = WSZ1 * NTILES * NPASS1
MG1 = 32
CHB1 = 1600


def _sc_agg1(src, dst, axr, rc1, x):
    @functools.partial(
        pl.kernel,
        mesh=_MESH,
        compiler_params=_SC_PARAMS,
        out_type=jax.ShapeDtypeStruct((NPAD1, H1 * D), jnp.float32),
        scratch_types=[
            pltpu.VMEM((WSZ1, H1 * D), jnp.float32),
            pltpu.VMEM((WSZ1, 16), jnp.float32),
            pltpu.VMEM((CHB1,), jnp.int32),
            pltpu.VMEM((CHB1,), jnp.int32),
            pltpu.VMEM((CHB1 + 3 * MG1,), jnp.int32),
            pltpu.VMEM((CHB1 + 3 * MG1,), jnp.int32),
            pltpu.VMEM((CHB1 + 3 * MG1,), jnp.int32),
            pltpu.VMEM((MG1, D), jnp.float32),
            pltpu.VMEM((MG1, 16), jnp.float32),
            pltpu.VMEM((16,), jnp.float32),
            pltpu.SemaphoreType.DMA,
            pltpu.SemaphoreType.DMA,
        ],
    )
    def k(src_hbm, dst_hbm, axr_hbm, rc_hbm, x_hbm, z_hbm,
          zf, rc_w, src_b, dst_b, m_src, m_dloc, m_eid, rows, arows, wtmp,
          sem1, sem2):
        wid = lax.axis_index("s") * 2 + lax.axis_index("c")
        zf16 = jnp.zeros((16,), jnp.float32)
        zi16 = jnp.zeros((16,), jnp.int32)
        lane = lax.iota(jnp.int32, 16)

        for p in range(NPASS1):
            lo = (p * NTILES + wid) * WSZ1

            @pl.loop(0, WSZ1)
            def _(r):
                @pl.loop(0, H1 * D, step=16)
                def _(q):
                    zf.at[r][pl.ds(q, 16)] = zf16

            pltpu.sync_copy(rc_hbm.at[pl.ds(lo, WSZ1)], rc_w)

            @pl.loop(0, E, step=CHB1)
            def _(cb):
                pltpu.sync_copy(src_hbm.at[pl.ds(cb, CHB1)], src_b)
                pltpu.sync_copy(dst_hbm.at[pl.ds(cb, CHB1)], dst_b)

                def scan_body(v, ptr):
                    dvec = dst_b[pl.ds(v * 16, 16)]
                    svec = src_b[pl.ds(v * 16, 16)]
                    evec = cb + v * 16 + lane
                    m = (dvec >= lo) & (dvec < lo + WSZ1)
                    plsc.store_compressed(m_src.at[pl.ds(ptr, 16)], svec, mask=m)
                    plsc.store_compressed(m_dloc.at[pl.ds(ptr, 16)],
                                          dvec - lo, mask=m)
                    plsc.store_compressed(m_eid.at[pl.ds(ptr, 16)], evec, mask=m)
                    return ptr + jnp.sum(m.astype(jnp.int32))

                ptr = lax.fori_loop(0, CHB1 // 16, scan_body, jnp.int32(0))

                t0 = (ptr // 16) * 16
                rel = ptr - t0
                keep = lane < rel
                m_src[pl.ds(t0, 16)] = jnp.where(keep,
                                                 m_src[pl.ds(t0, 16)], 0)
                m_dloc[pl.ds(t0, 16)] = jnp.where(keep,
                                                  m_dloc[pl.ds(t0, 16)], 0)
                m_eid[pl.ds(t0, 16)] = jnp.where(keep,
                                                 m_eid[pl.ds(t0, 16)], 0)

                @pl.loop(16, MG1 + 16, step=16)
                def _(t):
                    m_src[pl.ds(t0 + t, 16)] = zi16
                    m_dloc[pl.ds(t0 + t, 16)] = zi16
                    m_eid[pl.ds(t0 + t, 16)] = zi16

                def gbody(i, _):
                    g = i * MG1
                    c1 = pltpu.async_copy(
                        x_hbm.at[m_src.at[pl.ds(g, MG1)]], rows, sem1)
                    c2 = pltpu.async_copy(
                        axr_hbm.at[m_eid.at[pl.ds(g, MG1)]], arows, sem2)
                    c1.wait()
                    c2.wait()

                    def sgbody(v, _):
                        base = g + v * 16
                        dl_v = m_dloc[pl.ds(base, 16)]
                        for j in range(16):
                            @pl.when(base + j < ptr)
                            def _():
                                dloc = dl_v[j]
                                wv = (arows.at[v * 16 + j][pl.ds(0, 16)]
                                      * rc_w.at[dloc][pl.ds(0, 16)])
                                zrow = zf.at[dloc]
                                xrow = rows.at[v * 16 + j]
                                xv = [xrow[pl.ds(q * 16, 16)]
                                      for q in range(D // 16)]
                                for h in range(H1):
                                    wsc = wv[h]
                                    for q in range(D // 16):
                                        plsc.addupdate(
                                            zrow.at[pl.ds(h * D + q * 16, 16)],
                                            xv[q] * wsc)
                        return 0

                    nv = jnp.minimum((ptr - g + 15) // 16, MG1 // 16)
                    lax.fori_loop(0, nv, sgbody, 0)
                    return 0

                ng = (ptr + MG1 - 1) // MG1
                lax.fori_loop(0, ng, gbody, 0)

            pltpu.sync_copy(zf, z_hbm.at[pl.ds(lo, WSZ1)])

    return k(src, dst, axr, rc1, x)


def _leaky(x):
    return jnp.where(x >= 0.0, x, x * 0.2)


def kernel(x, edge_index, edge_attr, batch,
           W1, att_src1, att_dst1, We1, att_edge1, b1,
           W2, att_src2, att_dst2, We2, att_edge2, b2):
    src, dst = edge_index[0], edge_index[1]
    seg = jax.ops.segment_sum

    W1r = W1.reshape(D, H1, C1)
    Ms1 = jnp.einsum('dhc,hc->dh', W1r, att_src1[0])
    Md1 = jnp.einsum('dhc,hc->dh', W1r, att_dst1[0])
    Me1 = jnp.einsum('dhc,hc->dh', We1.reshape(DE, H1, C1), att_edge1[0])
    a_src = x @ Ms1
    a_dst = x @ Md1
    a_edge = edge_attr @ Me1
    axs, parts = [], []
    for h in range(H1):
        ax_h, part_h = _sc_edge_stats(src, dst, a_edge[:, h],
                                      a_src[:, h], a_dst[:, h])
        axs.append(ax_h)
        parts.append(part_h)
    deg = parts[0][:, 2, :].sum(axis=0)
    asum = jnp.stack([p[:, 0, :].sum(axis=0) for p in parts], axis=1)
    s_ae = jnp.stack([p[:, 1, :].sum(axis=0) for p in parts], axis=1)
    a_edge_loop = s_ae / jnp.clip(deg, 1.0)[:, None]
    aexp_loop = jnp.exp(_leaky(a_src + a_dst + a_edge_loop))
    asumT = asum + aexp_loop
    recip1 = 1.0 / (asumT + 1e-16)
    wl = aexp_loop * recip1
    axr = jnp.pad(jnp.stack(axs, axis=1), ((0, 0), (0, 16 - H1)))
    rc1 = jnp.pad(recip1, ((0, NPAD1 - N), (0, 16 - H1)))
    z1p = _sc_agg1(src, dst, axr, rc1, x)
    z = z1p[:N].reshape(N, H1, D) + wl[:, :, None] * x[:, None, :]
    h1 = jax.nn.elu(jnp.einsum('nhd,dhc->nhc', z, W1r).reshape(N, H1 * C1) + b1)

    a_src2 = h1 @ (W2 @ att_src2[0, 0])
    a_dst2 = h1 @ (W2 @ att_dst2[0, 0])
    a_edge2 = edge_attr @ (We2 @ att_edge2[0, 0])
    hp = h1 @ W2

    aexp2, part2 = _sc_edge_stats(src, dst, a_edge2, a_src2, a_dst2)
    asum2 = part2[:, 0, :].sum(axis=0)
    s_ae2 = part2[:, 1, :].sum(axis=0)

    ael2 = s_ae2 / jnp.clip(deg, 1.0)
    ael_exp = jnp.exp(_leaky(a_src2 + a_dst2 + ael2))
    asum2T = asum2 + ael_exp
    recip2 = 1.0 / (asum2T + 1e-16)
    wl2 = ael_exp * recip2
    recip2pad = jnp.concatenate([recip2, jnp.zeros((NPAD2 - N,), jnp.float32)])
    z2p = _sc_agg2(src, dst, aexp2, recip2pad, hp)
    z2 = z2p[:N] + wl2[:, None] * hp
    h2 = jax.nn.elu(z2 + b2)
    cnt = seg(jnp.ones((N,), h2.dtype), batch, num_segments=NG)
    pooled = seg(h2, batch, num_segments=NG) / jnp.clip(cnt, 1.0)[:, None]
    return pooled
